# baseline (speedup 1.0000x reference)
import functools

import numpy as np
import jax
import jax.numpy as jnp
from jax import lax
from jax.experimental import pallas as pl
from jax.experimental.pallas import tpu as pltpu
from jax.experimental.pallas import tpu_sc as plsc

N_NODES = 10000
N_EDGES = 320000
D = 128

NC = 2
NS = 16
NW = NC * NS
CHUNK = 64
CPW = 160
G = 8
NGRP = CPW // G
E_PAD = NW * CPW * CHUNK
NBUF = 4
REAL_GRPS = N_EDGES // (G * CHUNK)

ACC_ROWS = 10112
ROWS_PER_SUB = ACC_ROWS // NS


def _sc_agg(h, src_r, dst_r, src_p, dst_p, zc):
    mesh = plsc.VectorSubcoreMesh(core_axis_name="c", subcore_axis_name="s")

    @functools.partial(
        pl.kernel,
        out_type=jax.ShapeDtypeStruct((NC, ACC_ROWS, D), jnp.float32),
        mesh=mesh,
        scratch_types=[
            pltpu.VMEM((2, G, CHUNK), jnp.int32),
            pltpu.VMEM((2, G, CHUNK), jnp.int32),
            pltpu.VMEM((CHUNK, D), jnp.float32),
            pltpu.VMEM((CHUNK, D), jnp.float32),
            pltpu.VMEM((CHUNK, D), jnp.float32),
            pltpu.VMEM((CHUNK, D), jnp.float32),
            pltpu.VMEM_SHARED((ACC_ROWS, D), jnp.float32),
            pltpu.SemaphoreType.DMA,
            pltpu.SemaphoreType.DMA,
            pltpu.SemaphoreType.DMA,
            pltpu.SemaphoreType.DMA,
            pltpu.SemaphoreType.DMA,
            pltpu.SemaphoreType.DMA,
            pltpu.SemaphoreType.DMA,
        ],
    )
    def k(h_hbm, srcr_hbm, dstr_hbm, srcp_hbm, dstp_hbm, zc_hbm, out_hbm,
          sidx, didx, r0, r1, r2, r3, acc,
          isem0, isem1, rsem0, rsem1, rsem2, rsem3, zsem):
        c = lax.axis_index("c")
        s = lax.axis_index("s")
        wid = c * NS + s
        rows = (r0, r1, r2, r3)
        rsems = (rsem0, rsem1, rsem2, rsem3)
        isems = (isem0, isem1)
        base0 = wid * CPW
        real_rows = REAL_GRPS * G

        def idx_group_start(g, gb):
            gbase = base0 + g * G

            @pl.when(gbase < real_rows)
            def _():
                gb8 = pl.multiple_of(gbase, 8)
                pltpu.async_copy(srcr_hbm.at[pl.ds(gb8, G)], sidx.at[gb],
                                 isems[gb])
                pltpu.async_copy(dstr_hbm.at[pl.ds(gb8, G)], didx.at[gb],
                                 isems[gb])

            @pl.when(gbase >= real_rows)
            def _():
                pb8 = pl.multiple_of(gbase - real_rows, 8)
                pltpu.async_copy(srcp_hbm.at[pl.ds(pb8, G)], sidx.at[gb],
                                 isems[gb])
                pltpu.async_copy(dstp_hbm.at[pl.ds(pb8, G)], didx.at[gb],
                                 isems[gb])

        def idx_group_wait(g, gb):
            pltpu.make_async_copy(srcr_hbm.at[pl.ds(0, G)], sidx.at[gb],
                                  isems[gb]).wait()
            pltpu.make_async_copy(dstr_hbm.at[pl.ds(0, G)], didx.at[gb],
                                  isems[gb]).wait()

        def gather_start(gb, t, b):
            pltpu.async_copy(h_hbm.at[sidx.at[gb, t]], rows[b], rsems[b])

        def gather_wait(gb, t, b):
            pltpu.make_async_copy(h_hbm.at[sidx.at[gb, t]], rows[b],
                                  rsems[b]).wait()

        idx_group_start(0, 0)
        zbase = s * ROWS_PER_SUB
        zero_cp = pltpu.async_copy(zc_hbm, acc.at[pl.ds(zbase, ROWS_PER_SUB)],
                                   zsem)

        idx_group_wait(0, 0)
        for b in range(NBUF):
            gather_start(0, b, b)
        zero_cp.wait()
        plsc.subcore_barrier()

        @pl.loop(0, NGRP, step=2)
        def _(g0):
            for gb in range(2):
                g = g0 + gb

                @pl.when(g + 1 < NGRP)
                def _():
                    idx_group_start(g + 1, 1 - gb)

                for t in range(G):
                    b = t % NBUF
                    gather_wait(gb, t, b)
                    pltpu.sync_copy(rows[b], acc.at[didx.at[gb, t]], add=True)
                    if t + NBUF < G:
                        gather_start(gb, t + NBUF, b)
                    else:
                        if t == G - NBUF:
                            @pl.when(g + 1 < NGRP)
                            def _():
                                idx_group_wait(g + 1, 1 - gb)

                        @pl.when(g + 1 < NGRP)
                        def _():
                            gather_start(1 - gb, t + NBUF - G, b)

        plsc.subcore_barrier()

        obase = s * ROWS_PER_SUB
        pltpu.sync_copy(acc.at[pl.ds(obase, ROWS_PER_SUB)],
                        out_hbm.at[c, pl.ds(obase, ROWS_PER_SUB)])

    return k(h, src_r, dst_r, src_p, dst_p, zc)


BLK = 2000


def _mlp(p, h, W1, b1, W2, b2):
    def body(p_ref, h_ref, w1_ref, b1_ref, w2_ref, b2_ref, o_ref):
        z = p_ref[0] + p_ref[1] + h_ref[...]
        z = jnp.maximum(
            jnp.dot(z.astype(jnp.bfloat16), w1_ref[...],
                    preferred_element_type=jnp.float32) + b1_ref[...],
            0.0,
        )
        o_ref[...] = (
            jnp.dot(z.astype(jnp.bfloat16), w2_ref[...],
                    preferred_element_type=jnp.float32) + b2_ref[...]
        )

    return pl.pallas_call(
        body,
        grid=(N_NODES // BLK,),
        in_specs=[
            pl.BlockSpec((2, BLK, D), lambda i: (0, i, 0)),
            pl.BlockSpec((BLK, D), lambda i: (i, 0)),
            pl.BlockSpec((D, D), lambda i: (0, 0)),
            pl.BlockSpec((1, D), lambda i: (0, 0)),
            pl.BlockSpec((D, D), lambda i: (0, 0)),
            pl.BlockSpec((1, D), lambda i: (0, 0)),
        ],
        out_specs=pl.BlockSpec((BLK, D), lambda i: (i, 0)),
        out_shape=jax.ShapeDtypeStruct((N_NODES, D), jnp.float32),
    )(p, h, W1.astype(jnp.bfloat16), b1.reshape(1, D),
      W2.astype(jnp.bfloat16), b2.reshape(1, D))


_PAD = E_PAD - N_EDGES
_PAD_SRC = (np.arange(_PAD, dtype=np.int32) % N_NODES).reshape(-1, CHUNK)
_PAD_DST = (N_NODES + np.arange(_PAD, dtype=np.int32)
            % (ACC_ROWS - N_NODES)).reshape(-1, CHUNK)


def kernel(h, x, edge_index, W1_0, b1_0, W2_0, b2_0, W1_1, b1_1, W2_1, b2_1):
    src_r = edge_index[0].astype(jnp.int32).reshape(-1, CHUNK)
    dst_r = edge_index[1].astype(jnp.int32).reshape(-1, CHUNK)
    src_p = jnp.asarray(_PAD_SRC)
    dst_p = jnp.asarray(_PAD_DST)
    zc = jnp.zeros((ROWS_PER_SUB, D), jnp.float32)

    p1 = _sc_agg(h, src_r, dst_r, src_p, dst_p, zc)
    h1 = _mlp(p1, h, W1_0, b1_0, W2_0, b2_0)
    p2 = _sc_agg(h1, src_r, dst_r, src_p, dst_p, zc)
    h2 = _mlp(p2, h1, W1_1, b1_1, W2_1, b2_1)
    return (h2, x)


# speedup vs baseline: 1.0233x; 1.0233x over previous
import functools

import numpy as np
import jax
import jax.numpy as jnp
from jax import lax
from jax.experimental import pallas as pl
from jax.experimental.pallas import tpu as pltpu
from jax.experimental.pallas import tpu_sc as plsc

N_NODES = 10000
N_EDGES = 320000
D = 128

NC = 2
NS = 16
NW = NC * NS
CHUNK = 64
CPW = 160
G = 8
NGRP = CPW // G
E_PAD = NW * CPW * CHUNK
NBUF = 4
REAL_GRPS = N_EDGES // (G * CHUNK)

ACC_ROWS = 10112
ROWS_PER_SUB = ACC_ROWS // NS


def _sc_agg(h, src_r, dst_r, src_p, dst_p, zc):
    mesh = plsc.VectorSubcoreMesh(core_axis_name="c", subcore_axis_name="s")

    @functools.partial(
        pl.kernel,
        out_type=jax.ShapeDtypeStruct((NC, ACC_ROWS, D), jnp.float32),
        mesh=mesh,
        scratch_types=[
            pltpu.VMEM((2, G, CHUNK), jnp.int32),
            pltpu.VMEM((2, G, CHUNK), jnp.int32),
            pltpu.VMEM((CHUNK, D), jnp.float32),
            pltpu.VMEM((CHUNK, D), jnp.float32),
            pltpu.VMEM((CHUNK, D), jnp.float32),
            pltpu.VMEM((CHUNK, D), jnp.float32),
            pltpu.VMEM_SHARED((ACC_ROWS, D), jnp.float32),
            pltpu.SemaphoreType.DMA,
            pltpu.SemaphoreType.DMA,
            pltpu.SemaphoreType.DMA,
            pltpu.SemaphoreType.DMA,
            pltpu.SemaphoreType.DMA,
            pltpu.SemaphoreType.DMA,
            pltpu.SemaphoreType.DMA,
        ],
    )
    def k(h_hbm, srcr_hbm, dstr_hbm, srcp_hbm, dstp_hbm, zc_hbm, out_hbm,
          sidx, didx, r0, r1, r2, r3, acc,
          isem0, isem1, rsem0, rsem1, rsem2, rsem3, zsem):
        c = lax.axis_index("c")
        s = lax.axis_index("s")
        wid = c * NS + s
        rows = (r0, r1, r2, r3)
        rsems = (rsem0, rsem1, rsem2, rsem3)
        isems = (isem0, isem1)
        base0 = wid * CPW
        real_rows = REAL_GRPS * G

        def idx_group_start(g, gb):
            gbase = base0 + g * G

            @pl.when(gbase < real_rows)
            def _():
                gb8 = pl.multiple_of(gbase, 8)
                pltpu.async_copy(srcr_hbm.at[pl.ds(gb8, G)], sidx.at[gb],
                                 isems[gb])
                pltpu.async_copy(dstr_hbm.at[pl.ds(gb8, G)], didx.at[gb],
                                 isems[gb])

            @pl.when(gbase >= real_rows)
            def _():
                pb8 = pl.multiple_of(gbase - real_rows, 8)
                pltpu.async_copy(srcp_hbm.at[pl.ds(pb8, G)], sidx.at[gb],
                                 isems[gb])
                pltpu.async_copy(dstp_hbm.at[pl.ds(pb8, G)], didx.at[gb],
                                 isems[gb])

        def idx_group_wait(g, gb):
            pltpu.make_async_copy(srcr_hbm.at[pl.ds(0, G)], sidx.at[gb],
                                  isems[gb]).wait()
            pltpu.make_async_copy(dstr_hbm.at[pl.ds(0, G)], didx.at[gb],
                                  isems[gb]).wait()

        def gather_start(gb, t, b):
            pltpu.async_copy(h_hbm.at[sidx.at[gb, t]], rows[b], rsems[b])

        def gather_wait(gb, t, b):
            pltpu.make_async_copy(h_hbm.at[sidx.at[gb, t]], rows[b],
                                  rsems[b]).wait()

        idx_group_start(0, 0)

        @pl.loop(0, CHUNK)
        def _(i):
            @pl.loop(0, D, step=16)
            def _(j):
                r0[i, pl.ds(j, 16)] = jnp.zeros((16,), jnp.float32)

        zbase = s * ROWS_PER_SUB
        for off in range(0, ROWS_PER_SUB - CHUNK + 1, CHUNK):
            pltpu.sync_copy(r0, acc.at[pl.ds(zbase + off, CHUNK)])
        tail = ROWS_PER_SUB % CHUNK
        if tail:
            pltpu.sync_copy(r0.at[pl.ds(0, tail)],
                            acc.at[pl.ds(zbase + ROWS_PER_SUB - tail, tail)])

        idx_group_wait(0, 0)
        for b in range(NBUF):
            gather_start(0, b, b)
        plsc.subcore_barrier()

        @pl.loop(0, NGRP, step=2)
        def _(g0):
            for gb in range(2):
                g = g0 + gb

                @pl.when(g + 1 < NGRP)
                def _():
                    idx_group_start(g + 1, 1 - gb)

                for t in range(G):
                    b = t % NBUF
                    gather_wait(gb, t, b)
                    pltpu.sync_copy(rows[b], acc.at[didx.at[gb, t]], add=True)
                    if t + NBUF < G:
                        gather_start(gb, t + NBUF, b)
                    else:
                        if t == G - NBUF:
                            @pl.when(g + 1 < NGRP)
                            def _():
                                idx_group_wait(g + 1, 1 - gb)

                        @pl.when(g + 1 < NGRP)
                        def _():
                            gather_start(1 - gb, t + NBUF - G, b)

        plsc.subcore_barrier()

        obase = s * ROWS_PER_SUB
        pltpu.sync_copy(acc.at[pl.ds(obase, ROWS_PER_SUB)],
                        out_hbm.at[c, pl.ds(obase, ROWS_PER_SUB)])

    return k(h, src_r, dst_r, src_p, dst_p, zc)


BLK = 2000


def _mlp(p, h, W1, b1, W2, b2):
    def body(p_ref, h_ref, w1_ref, b1_ref, w2_ref, b2_ref, o_ref):
        z = p_ref[0] + p_ref[1] + h_ref[...]
        z = jnp.maximum(
            jnp.dot(z.astype(jnp.bfloat16), w1_ref[...],
                    preferred_element_type=jnp.float32) + b1_ref[...],
            0.0,
        )
        o_ref[...] = (
            jnp.dot(z.astype(jnp.bfloat16), w2_ref[...],
                    preferred_element_type=jnp.float32) + b2_ref[...]
        )

    return pl.pallas_call(
        body,
        grid=(N_NODES // BLK,),
        in_specs=[
            pl.BlockSpec((2, BLK, D), lambda i: (0, i, 0)),
            pl.BlockSpec((BLK, D), lambda i: (i, 0)),
            pl.BlockSpec((D, D), lambda i: (0, 0)),
            pl.BlockSpec((1, D), lambda i: (0, 0)),
            pl.BlockSpec((D, D), lambda i: (0, 0)),
            pl.BlockSpec((1, D), lambda i: (0, 0)),
        ],
        out_specs=pl.BlockSpec((BLK, D), lambda i: (i, 0)),
        out_shape=jax.ShapeDtypeStruct((N_NODES, D), jnp.float32),
    )(p, h, W1.astype(jnp.bfloat16), b1.reshape(1, D),
      W2.astype(jnp.bfloat16), b2.reshape(1, D))


_PAD = E_PAD - N_EDGES
_PAD_SRC = (np.arange(_PAD, dtype=np.int32) % N_NODES).reshape(-1, CHUNK)
_PAD_DST = (N_NODES + np.arange(_PAD, dtype=np.int32)
            % (ACC_ROWS - N_NODES)).reshape(-1, CHUNK)


def kernel(h, x, edge_index, W1_0, b1_0, W2_0, b2_0, W1_1, b1_1, W2_1, b2_1):
    src_r = edge_index[0].astype(jnp.int32).reshape(-1, CHUNK)
    dst_r = edge_index[1].astype(jnp.int32).reshape(-1, CHUNK)
    src_p = jnp.asarray(_PAD_SRC)
    dst_p = jnp.asarray(_PAD_DST)
    zc = jnp.zeros((ROWS_PER_SUB, D), jnp.float32)

    p1 = _sc_agg(h, src_r, dst_r, src_p, dst_p, zc)
    h1 = _mlp(p1, h, W1_0, b1_0, W2_0, b2_0)
    p2 = _sc_agg(h1, src_r, dst_r, src_p, dst_p, zc)
    h2 = _mlp(p2, h1, W1_1, b1_1, W2_1, b2_1)
    return (h2, x)


# speedup vs baseline: 1.0520x; 1.0280x over previous
import functools

import numpy as np
import jax
import jax.numpy as jnp
from jax import lax
from jax.experimental import pallas as pl
from jax.experimental.pallas import tpu as pltpu
from jax.experimental.pallas import tpu_sc as plsc

N_NODES = 10000
N_EDGES = 320000
D = 128

NC = 2
NS = 16
NW = NC * NS
CHUNK = 64
CPW = 160
G = 8
NGRP = CPW // G
E_PAD = NW * CPW * CHUNK
NBUF = 4
REAL_GRPS = N_EDGES // (G * CHUNK)

ACC_ROWS = 10112
ROWS_PER_SUB = ACC_ROWS // NS


def _sc_agg(h, src_r, dst_r, src_p, dst_p):
    mesh = plsc.VectorSubcoreMesh(core_axis_name="c", subcore_axis_name="s")

    @functools.partial(
        pl.kernel,
        out_type=jax.ShapeDtypeStruct((NC, ACC_ROWS, D), jnp.float32),
        mesh=mesh,
        scratch_types=[
            pltpu.VMEM((2, G, CHUNK), jnp.int32),
            pltpu.VMEM((2, G, CHUNK), jnp.int32),
            pltpu.VMEM((CHUNK, D), jnp.float32),
            pltpu.VMEM((CHUNK, D), jnp.float32),
            pltpu.VMEM((CHUNK, D), jnp.float32),
            pltpu.VMEM((CHUNK, D), jnp.float32),
            pltpu.VMEM_SHARED((ACC_ROWS, D), jnp.float32),
            pltpu.SemaphoreType.DMA,
            pltpu.SemaphoreType.DMA,
            pltpu.SemaphoreType.DMA,
            pltpu.SemaphoreType.DMA,
            pltpu.SemaphoreType.DMA,
            pltpu.SemaphoreType.DMA,
            pltpu.SemaphoreType.DMA,
        ],
    )
    def k(h_hbm, srcr_hbm, dstr_hbm, srcp_hbm, dstp_hbm, out_hbm,
          sidx, didx, r0, r1, r2, r3, acc,
          isem0, isem1, rsem0, rsem1, rsem2, rsem3, zsem):
        c = lax.axis_index("c")
        s = lax.axis_index("s")
        wid = c * NS + s
        rows = (r0, r1, r2, r3)
        rsems = (rsem0, rsem1, rsem2, rsem3)
        isems = (isem0, isem1)
        base0 = wid * CPW
        real_rows = REAL_GRPS * G

        def idx_group_start(g, gb):
            gbase = base0 + g * G

            @pl.when(gbase < real_rows)
            def _():
                gb8 = pl.multiple_of(gbase, 8)
                pltpu.async_copy(srcr_hbm.at[pl.ds(gb8, G)], sidx.at[gb],
                                 isems[gb])
                pltpu.async_copy(dstr_hbm.at[pl.ds(gb8, G)], didx.at[gb],
                                 isems[gb])

            @pl.when(gbase >= real_rows)
            def _():
                pb8 = pl.multiple_of(gbase - real_rows, 8)
                pltpu.async_copy(srcp_hbm.at[pl.ds(pb8, G)], sidx.at[gb],
                                 isems[gb])
                pltpu.async_copy(dstp_hbm.at[pl.ds(pb8, G)], didx.at[gb],
                                 isems[gb])

        def idx_group_wait(g, gb):
            pltpu.make_async_copy(srcr_hbm.at[pl.ds(0, G)], sidx.at[gb],
                                  isems[gb]).wait()
            pltpu.make_async_copy(dstr_hbm.at[pl.ds(0, G)], didx.at[gb],
                                  isems[gb]).wait()

        def gather_start(gb, t, b):
            pltpu.async_copy(h_hbm.at[sidx.at[gb, t]], rows[b], rsems[b])

        def gather_wait(gb, t, b):
            pltpu.make_async_copy(h_hbm.at[sidx.at[gb, t]], rows[b],
                                  rsems[b]).wait()

        idx_group_start(0, 0)

        @pl.loop(0, CHUNK)
        def _(i):
            @pl.loop(0, D, step=16)
            def _(j):
                r3[i, pl.ds(j, 16)] = jnp.zeros((16,), jnp.float32)

        zbase = s * ROWS_PER_SUB
        zoffs = list(range(0, ROWS_PER_SUB - CHUNK + 1, CHUNK))
        ztail = ROWS_PER_SUB % CHUNK
        for off in zoffs:
            pltpu.async_copy(r3, acc.at[pl.ds(zbase + off, CHUNK)], zsem)
        if ztail:
            pltpu.async_copy(r3.at[pl.ds(0, ztail)],
                             acc.at[pl.ds(zbase + ROWS_PER_SUB - ztail, ztail)],
                             zsem)

        idx_group_wait(0, 0)
        for b in range(NBUF - 1):
            gather_start(0, b, b)
        for off in zoffs:
            pltpu.make_async_copy(r3, acc.at[pl.ds(zbase + off, CHUNK)],
                                  zsem).wait()
        if ztail:
            pltpu.make_async_copy(r3.at[pl.ds(0, ztail)],
                                  acc.at[pl.ds(zbase + ROWS_PER_SUB - ztail,
                                               ztail)],
                                  zsem).wait()
        gather_start(0, NBUF - 1, NBUF - 1)
        plsc.subcore_barrier()

        @pl.loop(0, NGRP, step=2)
        def _(g0):
            for gb in range(2):
                g = g0 + gb

                @pl.when(g + 1 < NGRP)
                def _():
                    idx_group_start(g + 1, 1 - gb)

                for t in range(G):
                    b = t % NBUF
                    gather_wait(gb, t, b)
                    pltpu.sync_copy(rows[b], acc.at[didx.at[gb, t]], add=True)
                    if t + NBUF < G:
                        gather_start(gb, t + NBUF, b)
                    else:
                        if t == G - NBUF:
                            @pl.when(g + 1 < NGRP)
                            def _():
                                idx_group_wait(g + 1, 1 - gb)

                        @pl.when(g + 1 < NGRP)
                        def _():
                            gather_start(1 - gb, t + NBUF - G, b)

        plsc.subcore_barrier()

        obase = s * ROWS_PER_SUB
        pltpu.sync_copy(acc.at[pl.ds(obase, ROWS_PER_SUB)],
                        out_hbm.at[c, pl.ds(obase, ROWS_PER_SUB)])

    return k(h, src_r, dst_r, src_p, dst_p)


BLK = 2000


def _mlp(p, h, W1, b1, W2, b2):
    def body(p_ref, h_ref, w1_ref, b1_ref, w2_ref, b2_ref, o_ref):
        z = p_ref[0] + p_ref[1] + h_ref[...]
        z = jnp.maximum(
            jnp.dot(z, w1_ref[...], preferred_element_type=jnp.float32)
            + b1_ref[...],
            0.0,
        )
        o_ref[...] = (
            jnp.dot(z, w2_ref[...], preferred_element_type=jnp.float32)
            + b2_ref[...]
        )

    return pl.pallas_call(
        body,
        grid=(N_NODES // BLK,),
        in_specs=[
            pl.BlockSpec((2, BLK, D), lambda i: (0, i, 0)),
            pl.BlockSpec((BLK, D), lambda i: (i, 0)),
            pl.BlockSpec((D, D), lambda i: (0, 0)),
            pl.BlockSpec((1, D), lambda i: (0, 0)),
            pl.BlockSpec((D, D), lambda i: (0, 0)),
            pl.BlockSpec((1, D), lambda i: (0, 0)),
        ],
        out_specs=pl.BlockSpec((BLK, D), lambda i: (i, 0)),
        out_shape=jax.ShapeDtypeStruct((N_NODES, D), jnp.float32),
    )(p, h, W1, b1.reshape(1, D), W2, b2.reshape(1, D))


_PAD = E_PAD - N_EDGES
_PAD_SRC = (np.arange(_PAD, dtype=np.int32) % N_NODES).reshape(-1, CHUNK)
_PAD_DST = (N_NODES + np.arange(_PAD, dtype=np.int32)
            % (ACC_ROWS - N_NODES)).reshape(-1, CHUNK)


def kernel(h, x, edge_index, W1_0, b1_0, W2_0, b2_0, W1_1, b1_1, W2_1, b2_1):
    src_r = edge_index[0].astype(jnp.int32).reshape(-1, CHUNK)
    dst_r = edge_index[1].astype(jnp.int32).reshape(-1, CHUNK)
    src_p = jnp.asarray(_PAD_SRC)
    dst_p = jnp.asarray(_PAD_DST)

    p1 = _sc_agg(h, src_r, dst_r, src_p, dst_p)
    h1 = _mlp(p1, h, W1_0, b1_0, W2_0, b2_0)
    p2 = _sc_agg(h1, src_r, dst_r, src_p, dst_p)
    h2 = _mlp(p2, h1, W1_1, b1_1, W2_1, b2_1)
    return (h2, x)


# speedup vs baseline: 1.0993x; 1.0450x over previous
import functools

import numpy as np
import jax
import jax.numpy as jnp
from jax import lax
from jax.experimental import pallas as pl
from jax.experimental.pallas import tpu as pltpu
from jax.experimental.pallas import tpu_sc as plsc

N_NODES = 10000
N_EDGES = 320000
D = 128

NC = 2
NS = 16
NW = NC * NS
CHUNK = 64
CPW = 160
G = 8
NGRP = CPW // G
E_PAD = NW * CPW * CHUNK
NBUF = 4
REAL_GRPS = N_EDGES // (G * CHUNK)

ACC_ROWS = 10112
ROWS_PER_SUB = ACC_ROWS // NS


def _sc_agg(h, ei_r, src_p, dst_p):
    mesh = plsc.VectorSubcoreMesh(core_axis_name="c", subcore_axis_name="s")

    @functools.partial(
        pl.kernel,
        out_type=jax.ShapeDtypeStruct((NC, ACC_ROWS, D), jnp.float32),
        mesh=mesh,
        scratch_types=[
            pltpu.VMEM((2, G, CHUNK), jnp.int32),
            pltpu.VMEM((2, G, CHUNK), jnp.int32),
            pltpu.VMEM((CHUNK, D), jnp.float32),
            pltpu.VMEM((CHUNK, D), jnp.float32),
            pltpu.VMEM((CHUNK, D), jnp.float32),
            pltpu.VMEM((CHUNK, D), jnp.float32),
            pltpu.VMEM_SHARED((ACC_ROWS, D), jnp.float32),
            pltpu.SemaphoreType.DMA,
            pltpu.SemaphoreType.DMA,
            pltpu.SemaphoreType.DMA,
            pltpu.SemaphoreType.DMA,
            pltpu.SemaphoreType.DMA,
            pltpu.SemaphoreType.DMA,
            pltpu.SemaphoreType.DMA,
        ],
    )
    def k(h_hbm, ei_hbm, srcp_hbm, dstp_hbm, out_hbm,
          sidx, didx, r0, r1, r2, r3, acc,
          isem0, isem1, rsem0, rsem1, rsem2, rsem3, zsem):
        c = lax.axis_index("c")
        s = lax.axis_index("s")
        wid = c * NS + s
        rows = (r0, r1, r2, r3)
        rsems = (rsem0, rsem1, rsem2, rsem3)
        isems = (isem0, isem1)
        base0 = wid * CPW
        real_rows = REAL_GRPS * G

        def idx_group_start(g, gb):
            gbase = base0 + g * G

            @pl.when(gbase < real_rows)
            def _():
                gb8 = pl.multiple_of(gbase, 8)
                pltpu.async_copy(ei_hbm.at[0, pl.ds(gb8, G)], sidx.at[gb],
                                 isems[gb])
                pltpu.async_copy(ei_hbm.at[1, pl.ds(gb8, G)], didx.at[gb],
                                 isems[gb])

            @pl.when(gbase >= real_rows)
            def _():
                pb8 = pl.multiple_of(gbase - real_rows, 8)
                pltpu.async_copy(srcp_hbm.at[pl.ds(pb8, G)], sidx.at[gb],
                                 isems[gb])
                pltpu.async_copy(dstp_hbm.at[pl.ds(pb8, G)], didx.at[gb],
                                 isems[gb])

        def idx_group_wait(g, gb):
            pltpu.make_async_copy(ei_hbm.at[0, pl.ds(0, G)], sidx.at[gb],
                                  isems[gb]).wait()
            pltpu.make_async_copy(ei_hbm.at[1, pl.ds(0, G)], didx.at[gb],
                                  isems[gb]).wait()

        def gather_start(gb, t, b):
            pltpu.async_copy(h_hbm.at[sidx.at[gb, t]], rows[b], rsems[b])

        def gather_wait(gb, t, b):
            pltpu.make_async_copy(h_hbm.at[sidx.at[gb, t]], rows[b],
                                  rsems[b]).wait()

        idx_group_start(0, 0)

        @pl.loop(0, CHUNK)
        def _(i):
            @pl.loop(0, D, step=16)
            def _(j):
                r3[i, pl.ds(j, 16)] = jnp.zeros((16,), jnp.float32)

        zbase = s * ROWS_PER_SUB
        zoffs = list(range(0, ROWS_PER_SUB - CHUNK + 1, CHUNK))
        ztail = ROWS_PER_SUB % CHUNK
        for off in zoffs:
            pltpu.async_copy(r3, acc.at[pl.ds(zbase + off, CHUNK)], zsem)
        if ztail:
            pltpu.async_copy(r3.at[pl.ds(0, ztail)],
                             acc.at[pl.ds(zbase + ROWS_PER_SUB - ztail, ztail)],
                             zsem)

        idx_group_wait(0, 0)
        for b in range(NBUF - 1):
            gather_start(0, b, b)
        for off in zoffs:
            pltpu.make_async_copy(r3, acc.at[pl.ds(zbase + off, CHUNK)],
                                  zsem).wait()
        if ztail:
            pltpu.make_async_copy(r3.at[pl.ds(0, ztail)],
                                  acc.at[pl.ds(zbase + ROWS_PER_SUB - ztail,
                                               ztail)],
                                  zsem).wait()
        gather_start(0, NBUF - 1, NBUF - 1)
        plsc.subcore_barrier()

        @pl.loop(0, NGRP, step=2)
        def _(g0):
            for gb in range(2):
                g = g0 + gb

                @pl.when(g + 1 < NGRP)
                def _():
                    idx_group_start(g + 1, 1 - gb)

                for t in range(G):
                    b = t % NBUF
                    gather_wait(gb, t, b)
                    pltpu.sync_copy(rows[b], acc.at[didx.at[gb, t]], add=True)
                    if t + NBUF < G:
                        gather_start(gb, t + NBUF, b)
                    else:
                        if t == G - NBUF:
                            @pl.when(g + 1 < NGRP)
                            def _():
                                idx_group_wait(g + 1, 1 - gb)

                        @pl.when(g + 1 < NGRP)
                        def _():
                            gather_start(1 - gb, t + NBUF - G, b)

        plsc.subcore_barrier()

        obase = s * ROWS_PER_SUB
        pltpu.sync_copy(acc.at[pl.ds(obase, ROWS_PER_SUB)],
                        out_hbm.at[c, pl.ds(obase, ROWS_PER_SUB)])

    return k(h, ei_r, src_p, dst_p)


BLK = 2000


def _mlp(p, h, W1, b1, W2, b2):
    def body(p_ref, h_ref, w1_ref, b1_ref, w2_ref, b2_ref, o_ref):
        z = p_ref[0] + p_ref[1] + h_ref[...]
        z = jnp.maximum(
            jnp.dot(z, w1_ref[...], preferred_element_type=jnp.float32)
            + b1_ref[...],
            0.0,
        )
        o_ref[...] = (
            jnp.dot(z, w2_ref[...], preferred_element_type=jnp.float32)
            + b2_ref[...]
        )

    return pl.pallas_call(
        body,
        grid=(N_NODES // BLK,),
        in_specs=[
            pl.BlockSpec((2, BLK, D), lambda i: (0, i, 0)),
            pl.BlockSpec((BLK, D), lambda i: (i, 0)),
            pl.BlockSpec((D, D), lambda i: (0, 0)),
            pl.BlockSpec((1, D), lambda i: (0, 0)),
            pl.BlockSpec((D, D), lambda i: (0, 0)),
            pl.BlockSpec((1, D), lambda i: (0, 0)),
        ],
        out_specs=pl.BlockSpec((BLK, D), lambda i: (i, 0)),
        out_shape=jax.ShapeDtypeStruct((N_NODES, D), jnp.float32),
    )(p, h, W1, b1.reshape(1, D), W2, b2.reshape(1, D))


_PAD = E_PAD - N_EDGES
_PAD_SRC = (np.arange(_PAD, dtype=np.int32) % N_NODES).reshape(-1, CHUNK)
_PAD_DST = (N_NODES + np.arange(_PAD, dtype=np.int32)
            % (ACC_ROWS - N_NODES)).reshape(-1, CHUNK)


def kernel(h, x, edge_index, W1_0, b1_0, W2_0, b2_0, W1_1, b1_1, W2_1, b2_1):
    ei_r = edge_index.astype(jnp.int32).reshape(2, -1, CHUNK)
    src_p = jnp.asarray(_PAD_SRC)
    dst_p = jnp.asarray(_PAD_DST)

    p1 = _sc_agg(h, ei_r, src_p, dst_p)
    h1 = _mlp(p1, h, W1_0, b1_0, W2_0, b2_0)
    p2 = _sc_agg(h1, ei_r, src_p, dst_p)
    h2 = _mlp(p2, h1, W1_1, b1_1, W2_1, b2_1)
    return (h2, x)


# speedup vs baseline: 1.1013x; 1.0019x over previous
import functools

import numpy as np
import jax
import jax.numpy as jnp
from jax import lax
from jax.experimental import pallas as pl
from jax.experimental.pallas import tpu as pltpu
from jax.experimental.pallas import tpu_sc as plsc

N_NODES = 10000
N_EDGES = 320000
D = 128

NC = 2
NS = 16
NW = NC * NS
CHUNK = 64
CPW = 160
G = 8
NGRP = CPW // G
E_PAD = NW * CPW * CHUNK
NBUF = 4
REAL_GRPS = N_EDGES // (G * CHUNK)

ACC_ROWS = 10112
ROWS_PER_SUB = ACC_ROWS // NS


def _sc_agg(h, ei_r, src_p, dst_p):
    mesh = plsc.VectorSubcoreMesh(core_axis_name="c", subcore_axis_name="s")

    @functools.partial(
        pl.kernel,
        out_type=jax.ShapeDtypeStruct((NC, ACC_ROWS, D), jnp.float32),
        mesh=mesh,
        scratch_types=[
            pltpu.VMEM((2, G, CHUNK), jnp.int32),
            pltpu.VMEM((2, G, CHUNK), jnp.int32),
            pltpu.VMEM((CHUNK, D), jnp.float32),
            pltpu.VMEM((CHUNK, D), jnp.float32),
            pltpu.VMEM((CHUNK, D), jnp.float32),
            pltpu.VMEM((CHUNK, D), jnp.float32),
            pltpu.VMEM_SHARED((ACC_ROWS, D), jnp.float32),
            pltpu.SemaphoreType.DMA,
            pltpu.SemaphoreType.DMA,
            pltpu.SemaphoreType.DMA,
            pltpu.SemaphoreType.DMA,
            pltpu.SemaphoreType.DMA,
            pltpu.SemaphoreType.DMA,
            pltpu.SemaphoreType.DMA,
        ],
    )
    def k(h_hbm, ei_hbm, srcp_hbm, dstp_hbm, out_hbm,
          sidx, didx, r0, r1, r2, r3, acc,
          isem0, isem1, rsem0, rsem1, rsem2, rsem3, zsem):
        c = lax.axis_index("c")
        s = lax.axis_index("s")
        wid = c * NS + s
        rows = (r0, r1, r2, r3)
        rsems = (rsem0, rsem1, rsem2, rsem3)
        isems = (isem0, isem1)
        base0 = wid * CPW
        real_rows = REAL_GRPS * G

        def idx_group_start(g, gb):
            gbase = base0 + g * G

            @pl.when(gbase < real_rows)
            def _():
                gb8 = pl.multiple_of(gbase, 8)
                pltpu.async_copy(ei_hbm.at[0, pl.ds(gb8, G)], sidx.at[gb],
                                 isems[gb])
                pltpu.async_copy(ei_hbm.at[1, pl.ds(gb8, G)], didx.at[gb],
                                 isems[gb])

            @pl.when(gbase >= real_rows)
            def _():
                pb8 = pl.multiple_of(gbase - real_rows, 8)
                pltpu.async_copy(srcp_hbm.at[pl.ds(pb8, G)], sidx.at[gb],
                                 isems[gb])
                pltpu.async_copy(dstp_hbm.at[pl.ds(pb8, G)], didx.at[gb],
                                 isems[gb])

        def idx_group_wait(g, gb):
            pltpu.make_async_copy(ei_hbm.at[0, pl.ds(0, G)], sidx.at[gb],
                                  isems[gb]).wait()
            pltpu.make_async_copy(ei_hbm.at[1, pl.ds(0, G)], didx.at[gb],
                                  isems[gb]).wait()

        def gather_start(gb, t, b):
            pltpu.async_copy(h_hbm.at[sidx.at[gb, t]], rows[b], rsems[b])

        def gather_wait(gb, t, b):
            pltpu.make_async_copy(h_hbm.at[sidx.at[gb, t]], rows[b],
                                  rsems[b]).wait()

        idx_group_start(0, 0)

        @pl.loop(0, CHUNK)
        def _(i):
            @pl.loop(0, D, step=16)
            def _(j):
                r3[i, pl.ds(j, 16)] = jnp.zeros((16,), jnp.float32)

        zbase = s * ROWS_PER_SUB
        zoffs = list(range(0, ROWS_PER_SUB - CHUNK + 1, CHUNK))
        ztail = ROWS_PER_SUB % CHUNK
        for off in zoffs:
            pltpu.async_copy(r3, acc.at[pl.ds(zbase + off, CHUNK)], zsem)
        if ztail:
            pltpu.async_copy(r3.at[pl.ds(0, ztail)],
                             acc.at[pl.ds(zbase + ROWS_PER_SUB - ztail, ztail)],
                             zsem)

        idx_group_wait(0, 0)
        for b in range(NBUF - 1):
            gather_start(0, b, b)
        for off in zoffs:
            pltpu.make_async_copy(r3, acc.at[pl.ds(zbase + off, CHUNK)],
                                  zsem).wait()
        if ztail:
            pltpu.make_async_copy(r3.at[pl.ds(0, ztail)],
                                  acc.at[pl.ds(zbase + ROWS_PER_SUB - ztail,
                                               ztail)],
                                  zsem).wait()
        gather_start(0, NBUF - 1, NBUF - 1)
        plsc.subcore_barrier()

        @pl.loop(0, NGRP, step=2)
        def _(g0):
            for gb in range(2):
                g = g0 + gb

                @pl.when(g + 1 < NGRP)
                def _():
                    idx_group_start(g + 1, 1 - gb)

                for t in range(G):
                    b = t % NBUF
                    gather_wait(gb, t, b)
                    pltpu.sync_copy(rows[b], acc.at[didx.at[gb, t]], add=True)
                    if t + NBUF < G:
                        gather_start(gb, t + NBUF, b)
                    else:
                        if t == G - NBUF:
                            @pl.when(g + 1 < NGRP)
                            def _():
                                idx_group_wait(g + 1, 1 - gb)

                        @pl.when(g + 1 < NGRP)
                        def _():
                            gather_start(1 - gb, t + NBUF - G, b)

        plsc.subcore_barrier()

        obase = s * ROWS_PER_SUB
        pltpu.sync_copy(acc.at[pl.ds(obase, ROWS_PER_SUB)],
                        out_hbm.at[c, pl.ds(obase, ROWS_PER_SUB)])

    return k(h, ei_r, src_p, dst_p)


BLK = 2000


def _mlp(p, h, W1, b1, W2, b2):
    def body(p_ref, h_ref, w1_ref, b1_ref, w2_ref, b2_ref, o_ref):
        z = p_ref[0] + p_ref[1] + h_ref[...]
        z = jnp.maximum(
            jnp.dot(z, w1_ref[...], preferred_element_type=jnp.float32)
            + b1_ref[...],
            0.0,
        )
        o_ref[...] = (
            jnp.dot(z, w2_ref[...], preferred_element_type=jnp.float32)
            + b2_ref[...]
        )

    return pl.pallas_call(
        body,
        grid=(N_NODES // BLK,),
        compiler_params=pltpu.CompilerParams(
            dimension_semantics=("parallel",)),
        in_specs=[
            pl.BlockSpec((2, BLK, D), lambda i: (0, i, 0)),
            pl.BlockSpec((BLK, D), lambda i: (i, 0)),
            pl.BlockSpec((D, D), lambda i: (0, 0)),
            pl.BlockSpec((1, D), lambda i: (0, 0)),
            pl.BlockSpec((D, D), lambda i: (0, 0)),
            pl.BlockSpec((1, D), lambda i: (0, 0)),
        ],
        out_specs=pl.BlockSpec((BLK, D), lambda i: (i, 0)),
        out_shape=jax.ShapeDtypeStruct((N_NODES, D), jnp.float32),
    )(p, h, W1, b1.reshape(1, D), W2, b2.reshape(1, D))


_PAD = E_PAD - N_EDGES
_PAD_SRC = (np.arange(_PAD, dtype=np.int32) % N_NODES).reshape(-1, CHUNK)
_PAD_DST = (N_NODES + np.arange(_PAD, dtype=np.int32)
            % (ACC_ROWS - N_NODES)).reshape(-1, CHUNK)


def kernel(h, x, edge_index, W1_0, b1_0, W2_0, b2_0, W1_1, b1_1, W2_1, b2_1):
    ei_r = edge_index.astype(jnp.int32).reshape(2, -1, CHUNK)
    src_p = jnp.asarray(_PAD_SRC)
    dst_p = jnp.asarray(_PAD_DST)

    p1 = _sc_agg(h, ei_r, src_p, dst_p)
    h1 = _mlp(p1, h, W1_0, b1_0, W2_0, b2_0)
    p2 = _sc_agg(h1, ei_r, src_p, dst_p)
    h2 = _mlp(p2, h1, W1_1, b1_1, W2_1, b2_1)
    return (h2, x)
